# baseline (speedup 1.0000x reference)
import functools

import jax
import jax.numpy as jnp
from jax import lax
from jax.experimental import pallas as pl
from jax.experimental.pallas import tpu as pltpu
from jax.experimental.pallas import tpu_sc as plsc

B = 16384
V = 100000
D = 64

_NC = 2
_NS = 16
_NW = _NC * _NS
_BPW = B // _NW

_RCH = 160
_NRCH_TOTAL = V // _RCH

_GCH = 128
_NGCH = _BPW // _GCH


def _repack3(t_umlp, t_umf, t_uv, t_ua, t_imlp, t_imf):
    mesh = plsc.VectorSubcoreMesh(core_axis_name="c", subcore_axis_name="s")

    @functools.partial(
        pl.kernel,
        out_type=[jax.ShapeDtypeStruct((V, 128), jnp.float32)] * 3,
        mesh=mesh,
        scratch_types=[
            pltpu.VMEM((_RCH, 64), jnp.float32),
            pltpu.VMEM((_RCH, 64), jnp.float32),
            pltpu.VMEM((_RCH, 128), jnp.float32),
            pltpu.SemaphoreType.DMA,
            pltpu.SemaphoreType.DMA,
        ],
    )
    def k(ta0, tb0, ta1, tb1, ta2, tb2, o0, o1, o2, bA, bB, oc, sA, sB):
        wid = lax.axis_index("s") * _NC + lax.axis_index("c")
        nch = (_NRCH_TOTAL - wid + _NW - 1) // _NW
        for tA, tB, out in ((ta0, tb0, o0), (ta1, tb1, o1), (ta2, tb2, o2)):
            def body(c, carry, tA=tA, tB=tB, out=out):
                r0 = pl.multiple_of((wid + c * _NW) * _RCH, 8)
                cpA = pltpu.async_copy(tA.at[pl.ds(r0, _RCH)], bA, sA)
                cpB = pltpu.async_copy(tB.at[pl.ds(r0, _RCH)], bB, sB)
                cpA.wait()
                cpB.wait()

                def inter(j0, c2):
                    for dj in range(8):
                        j = j0 * 8 + dj
                        for q in range(4):
                            sl = pl.ds(16 * q, 16)
                            oc[j, sl] = bA[j, sl]
                            oc[j, pl.ds(64 + 16 * q, 16)] = bB[j, sl]
                    return c2

                lax.fori_loop(0, _RCH // 8, inter, None)
                pltpu.sync_copy(oc, out.at[pl.ds(r0, _RCH)])
                return carry

            lax.fori_loop(0, nch, body, None)

    return k(t_umlp, t_umf, t_uv, t_ua, t_imlp, t_imf)


def _gather3(uidx, iidx, p0, p1, p2):
    mesh = plsc.VectorSubcoreMesh(core_axis_name="c", subcore_axis_name="s")

    @functools.partial(
        pl.kernel,
        out_type=[jax.ShapeDtypeStruct((B, 128), jnp.float32)] * 3,
        mesh=mesh,
        scratch_types=[
            pltpu.VMEM((_BPW,), jnp.int32),
            pltpu.VMEM((_BPW,), jnp.int32),
            pltpu.VMEM((_GCH, 128), jnp.float32),
            pltpu.SemaphoreType.DMA,
        ],
    )
    def k(uh, ih, q0, q1, q2, o0, o1, o2, iu, ii, g0, s0):
        wid = lax.axis_index("s") * _NC + lax.axis_index("c")
        base = wid * _BPW
        pltpu.sync_copy(uh.at[pl.ds(base, _BPW)], iu)
        pltpu.sync_copy(ih.at[pl.ds(base, _BPW)], ii)
        for tab, idx, out in ((q0, iu, o0), (q1, iu, o1), (q2, ii, o2)):
            def body(c, carry, tab=tab, idx=idx, out=out):
                cp = pltpu.async_copy(
                    tab.at[idx.at[pl.ds(c * _GCH, _GCH)]], g0, s0)
                cp.wait()
                pltpu.sync_copy(g0, out.at[pl.ds(base + c * _GCH, _GCH)])
                return carry

            lax.fori_loop(0, _NGCH, body, None)

    return k(uidx, iidx, p0, p1, p2)


_BLK = 1024


def _poster_body(pb, few0, feb0, few1, feb1, out):
    dot = lambda a, b: lax.dot_general(a, b, (((1,), (0,)), ((), ())),
                                       preferred_element_type=jnp.float32)
    x = pb[...].astype(jnp.bfloat16)
    h = dot(x, few0[...]) + feb0[...]
    h = jnp.maximum(h, 0.0).astype(jnp.bfloat16)
    out[...] = dot(h, few1[...]) + feb1[...]


def _poster(poster, few0, feb0, few1, feb1):
    grid = (B // _BLK,)
    full = lambda a: pl.BlockSpec(a.shape, lambda i: (0,) * a.ndim)
    return pl.pallas_call(
        _poster_body,
        grid=grid,
        in_specs=[pl.BlockSpec((_BLK, 2048), lambda i: (i, 0)),
                  full(few0), full(feb0), full(few1), full(feb1)],
        out_specs=pl.BlockSpec((_BLK, D), lambda i: (i, 0)),
        out_shape=jax.ShapeDtypeStruct((B, D), jnp.float32),
    )(poster, few0, feb0, few1, feb1)


_FBLK = 2048


def _final_body(pe, gu1, gu2, gi,
                fcw0u, fcw0i, fcb0, fcw1, fcb1,
                fvw0u, fvw0p, fvb0, fvw1, fvb1,
                atw, atb, afw_mlp, afw_mf, afw_v, afb, out):
    dot = lambda a, b: lax.dot_general(a, b, (((1,), (0,)), ((), ())),
                                       preferred_element_type=jnp.float32)
    u1 = gu1[...]
    u2 = gu2[...]
    gi_ = gi[...]

    mlp = dot(u1, fcw0u[...]) + dot(gi_, fcw0i[...]) + fcb0[...]
    mlp = jnp.maximum(mlp, 0.0)
    mlp = jnp.maximum(dot(mlp, fcw1[...]) + fcb1[...], 0.0)

    v = dot(u2, fvw0u[...]) + dot(pe[...], fvw0p[...]) + fvb0[...]
    v = jnp.maximum(v, 0.0)
    v = jnp.maximum(dot(v, fvw1[...]) + fvb1[...], 0.0)

    att = jax.nn.sigmoid(dot(jnp.maximum(u2, 0.0), atw[...]) + atb[...])
    pre = (dot(mlp * att[:, 0:1], afw_mlp[...])
           + att[:, 1:2] * dot(u1 * gi_, afw_mf[...])
           + dot(v * att[:, 2:3], afw_v[...])
           + afb[...])
    out[...] = jax.nn.sigmoid(pre)


def _final(pe, gu1, gu2, gi,
           fcw0u, fcw0i, fcb0, fcw1, fcb1,
           fvw0u, fvw0p, fvb0, fvw1, fvb1,
           atw, atb, afw_mlp, afw_mf, afw_v, afb):
    grid = (B // _FBLK,)
    full = lambda a: pl.BlockSpec(a.shape, lambda i: (0,) * a.ndim)
    args = (pe, gu1, gu2, gi,
            fcw0u, fcw0i, fcb0, fcw1, fcb1,
            fvw0u, fvw0p, fvb0, fvw1, fvb1,
            atw, atb, afw_mlp, afw_mf, afw_v, afb)
    in_specs = ([pl.BlockSpec((_FBLK, D), lambda i: (i, 0))]
                + [pl.BlockSpec((_FBLK, 128), lambda i: (i, 0))] * 3
                + [full(a) for a in args[4:]])
    return pl.pallas_call(
        _final_body,
        grid=grid,
        in_specs=in_specs,
        out_specs=pl.BlockSpec((_FBLK, 1), lambda i: (i, 0)),
        out_shape=jax.ShapeDtypeStruct((B, 1), jnp.float32),
    )(*args)


def kernel(user_indices, item_indices, poster_embeddings, emb_user_mlp,
           emb_item_mlp, emb_user_mf, emb_item_mf, emb_user_v, emb_atten,
           fe_W0, fe_b0, fe_W1, fe_b1, fc_W0, fc_b0, fc_W1, fc_b1,
           fv_W0, fv_b0, fv_W1, fv_b1, at_W, at_b, af_W, af_b):
    f32 = jnp.float32
    bf16 = jnp.bfloat16
    p_u1, p_u2, p_i = _repack3(emb_user_mlp, emb_user_mf, emb_user_v,
                               emb_atten, emb_item_mlp, emb_item_mf)
    gu1, gu2, gi = _gather3(user_indices, item_indices, p_u1, p_u2, p_i)
    pe = _poster(poster_embeddings, fe_W0.astype(bf16), fe_b0.reshape(1, -1),
                 fe_W1.astype(bf16), fe_b1.reshape(1, -1))
    z64 = jnp.zeros((64, 64), f32)
    z3 = jnp.zeros((64, 3), f32)
    z1 = jnp.zeros((64, 1), f32)
    return _final(
        pe, gu1, gu2, gi,
        jnp.concatenate([fc_W0[:64], z64], 0),
        jnp.concatenate([fc_W0[64:], z64], 0),
        fc_b0.reshape(1, -1), fc_W1, fc_b1.reshape(1, -1),
        jnp.concatenate([fv_W0[:64], z64], 0),
        fv_W0[64:], fv_b0.reshape(1, -1), fv_W1, fv_b1.reshape(1, -1),
        jnp.concatenate([z3, at_W], 0), at_b.reshape(1, -1),
        af_W[:32], jnp.concatenate([z1, af_W[32:96]], 0), af_W[96:],
        af_b.reshape(1, -1))


# speedup vs baseline: 2.1752x; 2.1752x over previous
import functools

import jax
import jax.numpy as jnp
from jax import lax
from jax.experimental import pallas as pl
from jax.experimental.pallas import tpu as pltpu
from jax.experimental.pallas import tpu_sc as plsc

B = 16384
V = 100000
D = 64

_NC = 2
_NS = 16
_NW = _NC * _NS
_BPW = B // _NW

_GCH = 128
_NGCH = _BPW // _GCH


_RCW = 2560


def _repack_body(a0, b0, a1, b1, a2, b2, o0, o1, o2):
    for a, b, o in ((a0, b0, o0), (a1, b1, o1), (a2, b2, o2)):
        at = lax.transpose(a[...], (1, 0))
        bt = lax.transpose(b[...], (1, 0))
        o[...] = lax.concatenate([at, bt], 1)


def _repack3(t_umlp, t_umf, t_uv, t_ua, t_imlp, t_imf):
    tabs = [t.T for t in (t_umlp, t_umf, t_uv, t_ua, t_imlp, t_imf)]
    grid = ((V + _RCW - 1) // _RCW,)
    in_spec = pl.BlockSpec((D, _RCW), lambda i: (0, i))
    out_spec = pl.BlockSpec((_RCW, 128), lambda i: (i, 0))
    return pl.pallas_call(
        _repack_body,
        grid=grid,
        in_specs=[in_spec] * 6,
        out_specs=[out_spec] * 3,
        out_shape=[jax.ShapeDtypeStruct((V, 128), jnp.float32)] * 3,
    )(*tabs)


def _gather3(uidx, iidx, p0, p1, p2):
    mesh = plsc.VectorSubcoreMesh(core_axis_name="c", subcore_axis_name="s")

    @functools.partial(
        pl.kernel,
        out_type=[jax.ShapeDtypeStruct((B, 128), jnp.float32)] * 3,
        mesh=mesh,
        scratch_types=[
            pltpu.VMEM((_BPW,), jnp.int32),
            pltpu.VMEM((_BPW,), jnp.int32),
            pltpu.VMEM((_GCH, 128), jnp.float32),
            pltpu.SemaphoreType.DMA,
        ],
    )
    def k(uh, ih, q0, q1, q2, o0, o1, o2, iu, ii, g0, s0):
        wid = lax.axis_index("s") * _NC + lax.axis_index("c")
        base = wid * _BPW
        pltpu.sync_copy(uh.at[pl.ds(base, _BPW)], iu)
        pltpu.sync_copy(ih.at[pl.ds(base, _BPW)], ii)
        for tab, idx, out in ((q0, iu, o0), (q1, iu, o1), (q2, ii, o2)):
            def body(c, carry, tab=tab, idx=idx, out=out):
                cp = pltpu.async_copy(
                    tab.at[idx.at[pl.ds(c * _GCH, _GCH)]], g0, s0)
                cp.wait()
                pltpu.sync_copy(g0, out.at[pl.ds(base + c * _GCH, _GCH)])
                return carry

            lax.fori_loop(0, _NGCH, body, None)

    return k(uidx, iidx, p0, p1, p2)


_BLK = 1024


def _poster_body(pb, few0, feb0, few1, feb1, out):
    dot = lambda a, b: lax.dot_general(a, b, (((1,), (0,)), ((), ())),
                                       preferred_element_type=jnp.float32)
    x = pb[...].astype(jnp.bfloat16)
    h = dot(x, few0[...]) + feb0[...]
    h = jnp.maximum(h, 0.0).astype(jnp.bfloat16)
    out[...] = dot(h, few1[...]) + feb1[...]


def _poster(poster, few0, feb0, few1, feb1):
    grid = (B // _BLK,)
    full = lambda a: pl.BlockSpec(a.shape, lambda i: (0,) * a.ndim)
    return pl.pallas_call(
        _poster_body,
        grid=grid,
        in_specs=[pl.BlockSpec((_BLK, 2048), lambda i: (i, 0)),
                  full(few0), full(feb0), full(few1), full(feb1)],
        out_specs=pl.BlockSpec((_BLK, D), lambda i: (i, 0)),
        out_shape=jax.ShapeDtypeStruct((B, D), jnp.float32),
    )(poster, few0, feb0, few1, feb1)


_FBLK = 2048


def _final_body(pe, gu1, gu2, gi,
                fcw0u, fcw0i, fcb0, fcw1, fcb1,
                fvw0u, fvw0p, fvb0, fvw1, fvb1,
                atw, atb, afw_mlp, afw_mf, afw_v, afb, out):
    dot = lambda a, b: lax.dot_general(a, b, (((1,), (0,)), ((), ())),
                                       preferred_element_type=jnp.float32)
    u1 = gu1[...]
    u2 = gu2[...]
    gi_ = gi[...]

    mlp = dot(u1, fcw0u[...]) + dot(gi_, fcw0i[...]) + fcb0[...]
    mlp = jnp.maximum(mlp, 0.0)
    mlp = jnp.maximum(dot(mlp, fcw1[...]) + fcb1[...], 0.0)

    v = dot(u2, fvw0u[...]) + dot(pe[...], fvw0p[...]) + fvb0[...]
    v = jnp.maximum(v, 0.0)
    v = jnp.maximum(dot(v, fvw1[...]) + fvb1[...], 0.0)

    att = jax.nn.sigmoid(dot(jnp.maximum(u2, 0.0), atw[...]) + atb[...])
    pre = (dot(mlp * att[:, 0:1], afw_mlp[...])
           + att[:, 1:2] * dot(u1 * gi_, afw_mf[...])
           + dot(v * att[:, 2:3], afw_v[...])
           + afb[...])
    out[...] = jax.nn.sigmoid(pre)


def _final(pe, gu1, gu2, gi,
           fcw0u, fcw0i, fcb0, fcw1, fcb1,
           fvw0u, fvw0p, fvb0, fvw1, fvb1,
           atw, atb, afw_mlp, afw_mf, afw_v, afb):
    grid = (B // _FBLK,)
    full = lambda a: pl.BlockSpec(a.shape, lambda i: (0,) * a.ndim)
    args = (pe, gu1, gu2, gi,
            fcw0u, fcw0i, fcb0, fcw1, fcb1,
            fvw0u, fvw0p, fvb0, fvw1, fvb1,
            atw, atb, afw_mlp, afw_mf, afw_v, afb)
    in_specs = ([pl.BlockSpec((_FBLK, D), lambda i: (i, 0))]
                + [pl.BlockSpec((_FBLK, 128), lambda i: (i, 0))] * 3
                + [full(a) for a in args[4:]])
    return pl.pallas_call(
        _final_body,
        grid=grid,
        in_specs=in_specs,
        out_specs=pl.BlockSpec((_FBLK, 1), lambda i: (i, 0)),
        out_shape=jax.ShapeDtypeStruct((B, 1), jnp.float32),
    )(*args)


def kernel(user_indices, item_indices, poster_embeddings, emb_user_mlp,
           emb_item_mlp, emb_user_mf, emb_item_mf, emb_user_v, emb_atten,
           fe_W0, fe_b0, fe_W1, fe_b1, fc_W0, fc_b0, fc_W1, fc_b1,
           fv_W0, fv_b0, fv_W1, fv_b1, at_W, at_b, af_W, af_b):
    f32 = jnp.float32
    bf16 = jnp.bfloat16
    p_u1, p_u2, p_i = _repack3(emb_user_mlp, emb_user_mf, emb_user_v,
                               emb_atten, emb_item_mlp, emb_item_mf)
    gu1, gu2, gi = _gather3(user_indices, item_indices, p_u1, p_u2, p_i)
    pe = _poster(poster_embeddings, fe_W0.astype(bf16), fe_b0.reshape(1, -1),
                 fe_W1.astype(bf16), fe_b1.reshape(1, -1))
    z64 = jnp.zeros((64, 64), f32)
    z3 = jnp.zeros((64, 3), f32)
    z1 = jnp.zeros((64, 1), f32)
    return _final(
        pe, gu1, gu2, gi,
        jnp.concatenate([fc_W0[:64], z64], 0),
        jnp.concatenate([fc_W0[64:], z64], 0),
        fc_b0.reshape(1, -1), fc_W1, fc_b1.reshape(1, -1),
        jnp.concatenate([fv_W0[:64], z64], 0),
        fv_W0[64:], fv_b0.reshape(1, -1), fv_W1, fv_b1.reshape(1, -1),
        jnp.concatenate([z3, at_W], 0), at_b.reshape(1, -1),
        af_W[:32], jnp.concatenate([z1, af_W[32:96]], 0), af_W[96:],
        af_b.reshape(1, -1))


# speedup vs baseline: 2.1851x; 1.0045x over previous
import functools

import jax
import jax.numpy as jnp
from jax import lax
from jax.experimental import pallas as pl
from jax.experimental.pallas import tpu as pltpu
from jax.experimental.pallas import tpu_sc as plsc

B = 16384
V = 100000
D = 64

_NC = 2
_NS = 16
_NW = _NC * _NS
_BPW = B // _NW

_GCH = 128
_NGCH = _BPW // _GCH


_RCW = 2560


def _repack_body(a0, b0, a1, b1, a2, b2, o0, o1, o2):
    eye = jnp.eye(D, dtype=jnp.float32)
    mxu_t = lambda x: lax.dot_general(x, eye, (((0,), (0,)), ((), ())),
                                      preferred_element_type=jnp.float32)
    for a, b, o in ((a0, b0, o0), (a1, b1, o1), (a2, b2, o2)):
        at = lax.transpose(a[...], (1, 0))
        bt = mxu_t(b[...])
        o[...] = lax.concatenate([at, bt], 1)


def _repack3(t_umlp, t_umf, t_uv, t_ua, t_imlp, t_imf):
    tabs = [t.T for t in (t_umlp, t_umf, t_uv, t_ua, t_imlp, t_imf)]
    grid = ((V + _RCW - 1) // _RCW,)
    in_spec = pl.BlockSpec((D, _RCW), lambda i: (0, i))
    out_spec = pl.BlockSpec((_RCW, 128), lambda i: (i, 0))
    return pl.pallas_call(
        _repack_body,
        grid=grid,
        in_specs=[in_spec] * 6,
        out_specs=[out_spec] * 3,
        out_shape=[jax.ShapeDtypeStruct((V, 128), jnp.float32)] * 3,
    )(*tabs)


def _gather3(uidx, iidx, p0, p1, p2):
    mesh = plsc.VectorSubcoreMesh(core_axis_name="c", subcore_axis_name="s")

    @functools.partial(
        pl.kernel,
        out_type=[jax.ShapeDtypeStruct((B, 128), jnp.float32)] * 3,
        mesh=mesh,
        scratch_types=[
            pltpu.VMEM((_BPW,), jnp.int32),
            pltpu.VMEM((_BPW,), jnp.int32),
            pltpu.VMEM((_GCH, 128), jnp.float32),
            pltpu.VMEM((_GCH, 128), jnp.float32),
            pltpu.SemaphoreType.DMA,
            pltpu.SemaphoreType.DMA,
        ],
    )
    def k(uh, ih, q0, q1, q2, o0, o1, o2, iu, ii, g0, g1, s0, s1):
        wid = lax.axis_index("s") * _NC + lax.axis_index("c")
        base = wid * _BPW
        pltpu.sync_copy(uh.at[pl.ds(base, _BPW)], iu)
        pltpu.sync_copy(ih.at[pl.ds(base, _BPW)], ii)
        steps = [(tab, idx, out, c)
                 for tab, idx, out in ((q0, iu, o0), (q1, iu, o1), (q2, ii, o2))
                 for c in range(_NGCH)]
        bufs = (g0, g1)
        sems = (s0, s1)

        def fire(i):
            tab, idx, _, c = steps[i]
            return pltpu.async_copy(
                tab.at[idx.at[pl.ds(c * _GCH, _GCH)]], bufs[i % 2],
                sems[i % 2])

        cps = {0: fire(0), 1: fire(1)}
        for i in range(len(steps)):
            _, _, out, c = steps[i]
            cps[i].wait()
            pltpu.sync_copy(bufs[i % 2], out.at[pl.ds(base + c * _GCH, _GCH)])
            if i + 2 < len(steps):
                cps[i + 2] = fire(i + 2)

    return k(uidx, iidx, p0, p1, p2)


_BLK = 2048


def _poster_body(pb, few0, feb0, few1, feb1, out):
    dot = lambda a, b: lax.dot_general(a, b, (((1,), (0,)), ((), ())),
                                       preferred_element_type=jnp.float32)
    x = pb[...].astype(jnp.bfloat16)
    h = dot(x, few0[...]) + feb0[...]
    h = jnp.maximum(h, 0.0).astype(jnp.bfloat16)
    out[...] = dot(h, few1[...]) + feb1[...]


def _poster(poster, few0, feb0, few1, feb1):
    grid = (B // _BLK,)
    full = lambda a: pl.BlockSpec(a.shape, lambda i: (0,) * a.ndim)
    return pl.pallas_call(
        _poster_body,
        grid=grid,
        in_specs=[pl.BlockSpec((_BLK, 2048), lambda i: (i, 0)),
                  full(few0), full(feb0), full(few1), full(feb1)],
        out_specs=pl.BlockSpec((_BLK, D), lambda i: (i, 0)),
        out_shape=jax.ShapeDtypeStruct((B, D), jnp.float32),
    )(poster, few0, feb0, few1, feb1)


_FBLK = 2048


def _final_body(pe, gu1, gu2, gi,
                fcw0u, fcw0i, fcb0, fcw1, fcb1,
                fvw0u, fvw0p, fvb0, fvw1, fvb1,
                atw, atb, afw_mlp, afw_mf, afw_v, afb, out):
    dot = lambda a, b: lax.dot_general(a, b, (((1,), (0,)), ((), ())),
                                       preferred_element_type=jnp.float32)
    u1 = gu1[...]
    u2 = gu2[...]
    gi_ = gi[...]

    mlp = dot(u1, fcw0u[...]) + dot(gi_, fcw0i[...]) + fcb0[...]
    mlp = jnp.maximum(mlp, 0.0)
    mlp = jnp.maximum(dot(mlp, fcw1[...]) + fcb1[...], 0.0)

    v = dot(u2, fvw0u[...]) + dot(pe[...], fvw0p[...]) + fvb0[...]
    v = jnp.maximum(v, 0.0)
    v = jnp.maximum(dot(v, fvw1[...]) + fvb1[...], 0.0)

    att = jax.nn.sigmoid(dot(jnp.maximum(u2, 0.0), atw[...]) + atb[...])
    pre = (dot(mlp * att[:, 0:1], afw_mlp[...])
           + att[:, 1:2] * dot(u1 * gi_, afw_mf[...])
           + dot(v * att[:, 2:3], afw_v[...])
           + afb[...])
    out[...] = jax.nn.sigmoid(pre)


def _final(pe, gu1, gu2, gi,
           fcw0u, fcw0i, fcb0, fcw1, fcb1,
           fvw0u, fvw0p, fvb0, fvw1, fvb1,
           atw, atb, afw_mlp, afw_mf, afw_v, afb):
    grid = (B // _FBLK,)
    full = lambda a: pl.BlockSpec(a.shape, lambda i: (0,) * a.ndim)
    args = (pe, gu1, gu2, gi,
            fcw0u, fcw0i, fcb0, fcw1, fcb1,
            fvw0u, fvw0p, fvb0, fvw1, fvb1,
            atw, atb, afw_mlp, afw_mf, afw_v, afb)
    in_specs = ([pl.BlockSpec((_FBLK, D), lambda i: (i, 0))]
                + [pl.BlockSpec((_FBLK, 128), lambda i: (i, 0))] * 3
                + [full(a) for a in args[4:]])
    return pl.pallas_call(
        _final_body,
        grid=grid,
        in_specs=in_specs,
        out_specs=pl.BlockSpec((_FBLK, 1), lambda i: (i, 0)),
        out_shape=jax.ShapeDtypeStruct((B, 1), jnp.float32),
    )(*args)


def kernel(user_indices, item_indices, poster_embeddings, emb_user_mlp,
           emb_item_mlp, emb_user_mf, emb_item_mf, emb_user_v, emb_atten,
           fe_W0, fe_b0, fe_W1, fe_b1, fc_W0, fc_b0, fc_W1, fc_b1,
           fv_W0, fv_b0, fv_W1, fv_b1, at_W, at_b, af_W, af_b):
    f32 = jnp.float32
    bf16 = jnp.bfloat16
    p_u1, p_u2, p_i = _repack3(emb_user_mlp, emb_user_mf, emb_user_v,
                               emb_atten, emb_item_mlp, emb_item_mf)
    gu1, gu2, gi = _gather3(user_indices, item_indices, p_u1, p_u2, p_i)
    pe = _poster(poster_embeddings, fe_W0.astype(bf16), fe_b0.reshape(1, -1),
                 fe_W1.astype(bf16), fe_b1.reshape(1, -1))
    z64 = jnp.zeros((64, 64), f32)
    z3 = jnp.zeros((64, 3), f32)
    z1 = jnp.zeros((64, 1), f32)
    return _final(
        pe, gu1, gu2, gi,
        jnp.concatenate([fc_W0[:64], z64], 0),
        jnp.concatenate([fc_W0[64:], z64], 0),
        fc_b0.reshape(1, -1), fc_W1, fc_b1.reshape(1, -1),
        jnp.concatenate([fv_W0[:64], z64], 0),
        fv_W0[64:], fv_b0.reshape(1, -1), fv_W1, fv_b1.reshape(1, -1),
        jnp.concatenate([z3, at_W], 0), at_b.reshape(1, -1),
        af_W[:32], jnp.concatenate([z1, af_W[32:96]], 0), af_W[96:],
        af_b.reshape(1, -1))
